# baseline (speedup 1.0000x reference)
import functools

import jax
import jax.numpy as jnp
from jax import lax
from jax.experimental import pallas as pl
from jax.experimental.pallas import tpu as pltpu
from jax.experimental.pallas import tpu_sc as plsc

B = 4096
V = 100000
NCC = 4
D = 512
SEQ = 77
P = 5
S = SEQ - P - NCC
L = 16

H = 16
BODY = SEQ - H

NC, NS = 2, 16
NW = NC * NS
BPW = B // NW
CHUNK = 16
NCHUNK = BPW // CHUNK


def _vcopy_row(dst, drow, src, srow):
    for l in range(D // L):
        dst[drow, pl.ds(l * L, L)] = src[srow, pl.ds(l * L, L)]


def _sc_assemble(label, cls_ctx, prefix, suffix):
    mesh = plsc.VectorSubcoreMesh(core_axis_name="c", subcore_axis_name="s")

    @functools.partial(
        pl.kernel,
        out_type=jax.ShapeDtypeStruct((B, SEQ, D), jnp.float32),
        mesh=mesh,
        scratch_types=[
            pltpu.VMEM((BPW,), jnp.int32),
            pltpu.VMEM((CHUNK, NCC, D), jnp.float32),
            pltpu.VMEM((H, D), jnp.float32),
            pltpu.VMEM((H, D), jnp.float32),
            pltpu.VMEM((BODY, D), jnp.float32),
            pltpu.VMEM((S, D), jnp.float32),
            pltpu.SemaphoreType.DMA,
            pltpu.SemaphoreType.DMA,
            pltpu.SemaphoreType.DMA,
            pltpu.SemaphoreType.DMA,
            pltpu.SemaphoreType.DMA,
        ],
    )
    def run(label_hbm, cls_hbm, pre_hbm, suf_hbm, out_hbm,
            idx_v, gbuf, head0, head1, body, stage,
            gsem, wh0, wh1, wb0, wb1):
        wid = lax.axis_index("s") * NC + lax.axis_index("c")
        base = wid * BPW

        pltpu.sync_copy(label_hbm.at[pl.ds(base, BPW)], idx_v)
        pltpu.sync_copy(suf_hbm.at[0], stage)
        heads = (head0, head1)
        for head in heads:
            pltpu.sync_copy(pre_hbm.at[0], head.at[pl.ds(0, P)])

        def init_head(j, _):
            _vcopy_row(head0, P + NCC + j, stage, j)
            _vcopy_row(head1, P + NCC + j, stage, j)
            return 0

        def init_body(j, _):
            _vcopy_row(body, j, stage, H - P - NCC + j)
            return 0

        lax.fori_loop(0, H - P - NCC, init_head, 0)
        lax.fori_loop(0, BODY, init_body, 0)

        whs = (wh0, wh1)
        wbs = (wb0, wb1)

        def head_cp(r, p):
            return pltpu.make_async_copy(
                heads[p], out_hbm.at[base + r, pl.ds(0, H), :], whs[p])

        def body_cp(r, p):
            return pltpu.make_async_copy(
                body, out_hbm.at[base + r, pl.ds(H, BODY), :], wbs[p])

        def chunk_step(c, _):
            off = pl.multiple_of(c * CHUNK, CHUNK)
            gcp = pltpu.make_async_copy(
                cls_hbm.at[idx_v.at[pl.ds(off, CHUNK)]], gbuf, gsem)
            gcp.start()
            gcp.wait()

            def row_pair(k, _):
                j0 = 2 * k
                for p in (0, 1):
                    j = j0 + p
                    r = c * CHUNK + j
                    first = jnp.logical_and(c == 0, k == 0)
                    @pl.when(jnp.logical_not(first))
                    def _():
                        head_cp(r - 2, p).wait()
                        body_cp(r - 2, p).wait()
                    for jj in range(NCC):
                        for l in range(D // L):
                            heads[p][P + jj, pl.ds(l * L, L)] = (
                                gbuf[j, jj, pl.ds(l * L, L)])
                    head_cp(r, p).start()
                    body_cp(r, p).start()
                return 0

            lax.fori_loop(0, CHUNK // 2, row_pair, 0)
            return 0

        lax.fori_loop(0, NCHUNK, chunk_step, 0)
        head_cp(BPW - 2, 0).wait()
        body_cp(BPW - 2, 0).wait()
        head_cp(BPW - 1, 1).wait()
        body_cp(BPW - 1, 1).wait()

    return run(label, cls_ctx, prefix, suffix)


def kernel(label, cls_ctx, token_prefix, token_suffix):
    return _sc_assemble(label.astype(jnp.int32), cls_ctx,
                        token_prefix, token_suffix)


# speedup vs baseline: 1.0066x; 1.0066x over previous
import functools

import jax
import jax.numpy as jnp
from jax import lax
from jax.experimental import pallas as pl
from jax.experimental.pallas import tpu as pltpu
from jax.experimental.pallas import tpu_sc as plsc

B = 4096
V = 100000
NCC = 4
D = 512
SEQ = 77
P = 5
S = SEQ - P - NCC
L = 16

H = 16
BODY = SEQ - H
SH = H - P - NCC

NC, NS = 2, 16
NW = NC * NS
BPW = B // NW
CHUNK = 16
NCHUNK = BPW // CHUNK
NHB = 4


def _sc_heads(label, cls_ctx, prefix, suffix):
    mesh = plsc.VectorSubcoreMesh(core_axis_name="c", subcore_axis_name="s")

    @functools.partial(
        pl.kernel,
        out_type=jax.ShapeDtypeStruct((B, SEQ, D), jnp.float32),
        mesh=mesh,
        scratch_types=[
            pltpu.VMEM((BPW,), jnp.int32),
            pltpu.VMEM((CHUNK, NCC, D), jnp.float32),
            pltpu.VMEM((CHUNK, NCC, D), jnp.float32),
            pltpu.VMEM((H, D), jnp.float32),
            pltpu.VMEM((H, D), jnp.float32),
            pltpu.VMEM((H, D), jnp.float32),
            pltpu.VMEM((H, D), jnp.float32),
            pltpu.VMEM((8, D), jnp.float32),
            pltpu.SemaphoreType.DMA,
            pltpu.SemaphoreType.DMA,
            pltpu.SemaphoreType.DMA,
            pltpu.SemaphoreType.DMA,
            pltpu.SemaphoreType.DMA,
            pltpu.SemaphoreType.DMA,
        ],
    )
    def run(label_hbm, cls_hbm, pre_hbm, suf_hbm, out_hbm,
            idx_v, gbuf0, gbuf1, h0, h1, h2, h3, stage,
            gs0, gs1, w0, w1, w2, w3):
        wid = lax.axis_index("s") * NC + lax.axis_index("c")
        base = wid * BPW
        heads = (h0, h1, h2, h3)
        gbufs = (gbuf0, gbuf1)
        gsems = (gs0, gs1)
        wsems = (w0, w1, w2, w3)

        pltpu.sync_copy(label_hbm.at[pl.ds(base, BPW)], idx_v)
        pltpu.sync_copy(suf_hbm.at[0, pl.ds(0, 8), :], stage)
        for head in heads:
            pltpu.sync_copy(pre_hbm.at[0], head.at[pl.ds(0, P)])

        def init_head(j, _):
            for head in heads:
                for l in range(D // L):
                    head[P + NCC + j, pl.ds(l * L, L)] = stage[j, pl.ds(l * L, L)]
            return 0

        lax.fori_loop(0, SH, init_head, 0)

        def gather_cp(c, q):
            off = pl.multiple_of(c * CHUNK, CHUNK)
            return pltpu.make_async_copy(
                cls_hbm.at[idx_v.at[pl.ds(off, CHUNK)]], gbufs[q], gsems[q])

        def head_cp(r, p):
            return pltpu.make_async_copy(
                heads[p], out_hbm.at[base + r, pl.ds(0, H), :], wsems[p])

        gather_cp(0, 0).start()

        def chunk_pair(t, _):
            for qc in (0, 1):
                c = 2 * t + qc
                gather_cp(c, qc).wait()

                @pl.when(c + 1 < NCHUNK)
                def _():
                    gather_cp(c + 1, 1 - qc).start()

                def row_quad(k, _):
                    for p in (0, 1, 2, 3):
                        j = 4 * k + p
                        r = c * CHUNK + j

                        @pl.when(r >= NHB)
                        def _():
                            head_cp(r - NHB, p).wait()

                        for jj in range(NCC):
                            for l in range(D // L):
                                heads[p][P + jj, pl.ds(l * L, L)] = (
                                    gbufs[qc][j, jj, pl.ds(l * L, L)])
                        head_cp(r, p).start()
                    return 0

                lax.fori_loop(0, CHUNK // 4, row_quad, 0)
            return 0

        lax.fori_loop(0, NCHUNK // 2, chunk_pair, 0)
        for p in (0, 1, 2, 3):
            head_cp(BPW - NHB + p, p).wait()

    return run(label, cls_ctx, prefix, suffix)


BB = 128


def _tc_body(suffix, out0):
    def body(suf_ref, out_in, out_hbm, slab, sem):
        del out_in
        i = pl.program_id(0)

        @pl.when(i == 0)
        def _():
            slab[...] = jnp.broadcast_to(
                suf_ref[0, SH:S, :][None], (BB, BODY, D))

        def cp(j):
            return pltpu.make_async_copy(
                slab, out_hbm.at[pl.ds(j * BB, BB), pl.ds(H, BODY), :], sem)

        cp(i).start()

        @pl.when(i > 0)
        def _():
            cp(i - 1).wait()

        @pl.when(i == B // BB - 1)
        def _():
            cp(i).wait()

    return pl.pallas_call(
        body,
        grid=(B // BB,),
        in_specs=[
            pl.BlockSpec((1, S, D), lambda i: (0, 0, 0)),
            pl.BlockSpec(memory_space=pl.ANY),
        ],
        out_specs=pl.BlockSpec(memory_space=pl.ANY),
        out_shape=jax.ShapeDtypeStruct((B, SEQ, D), jnp.float32),
        scratch_shapes=[
            pltpu.VMEM((BB, BODY, D), jnp.float32),
            pltpu.SemaphoreType.DMA,
        ],
        input_output_aliases={1: 0},
    )(suffix, out0)


def kernel(label, cls_ctx, token_prefix, token_suffix):
    out0 = _sc_heads(label.astype(jnp.int32), cls_ctx,
                     token_prefix, token_suffix)
    return _tc_body(token_suffix, out0)
